# baseline (speedup 1.0000x reference)
import functools

import jax
import jax.numpy as jnp
import numpy as np
from jax import lax
from jax.experimental import pallas as pl
from jax.experimental.pallas import tpu as pltpu
from jax.experimental.pallas import tpu_sc as plsc

EMBED = 128
T = 4
LANES = 16
CHUNKS = EMBED // LANES
UNROLL = 2


@functools.lru_cache(maxsize=None)
def _poe_factors(ctx, emb):
    nlo = 64
    nhi = ctx // nlo
    j = np.arange(emb)
    w = 10000.0 ** (-(j - (j % 2)) / emb)
    even = (j % 2 == 0)
    ah = (np.arange(nhi)[:, None] * nlo) * w[None, :]
    al = np.arange(nlo)[:, None] * w[None, :]
    u = np.where(even, np.sin(ah), np.cos(ah))
    v = np.where(even, np.cos(ah), -np.sin(ah))
    cl = np.cos(al)
    sl = np.sin(al)
    f32 = lambda a: jnp.asarray(a.astype(np.float32))
    return f32(u), f32(v), f32(cl), f32(sl)


def _poe(ctx, emb):
    u, v, cl, sl = _poe_factors(ctx, emb)
    poe3 = u[:, None, :] * cl[None, :, :] + v[:, None, :] * sl[None, :, :]
    return poe3.reshape(ctx * emb)


def _rsqrt_newton(v):
    i = lax.bitcast_convert_type(v, jnp.int32)
    i = jnp.int32(0x5F3759DF) - lax.shift_right_logical(i, 1)
    y = lax.bitcast_convert_type(i, jnp.float32)
    y = y * (1.5 - 0.5 * v * y * y)
    return y


def _shuffle(v, k):
    iota = lax.iota(jnp.int32, LANES)
    dnums = lax.GatherDimensionNumbers(
        offset_dims=(), collapsed_slice_dims=(0,), start_index_map=(0,))
    idx = jnp.bitwise_xor(iota, k)
    return lax.gather(v, idx[:, None], dimension_numbers=dnums,
                      slice_sizes=(1,),
                      mode=lax.GatherScatterMode.PROMISE_IN_BOUNDS)


def _hsum(v):
    for k in (8, 4, 2, 1):
        v = v + _shuffle(v, k)
    return v


def _make_sc_kernel(seq, num_cores, rows_per_w):
    mesh = plsc.VectorSubcoreMesh(core_axis_name="c", subcore_axis_name="s")

    @functools.partial(
        pl.kernel,
        mesh=mesh,
        out_type=jax.ShapeDtypeStruct((T, seq, EMBED), jnp.float32),
        scratch_types=[
            pltpu.VMEM((rows_per_w,), jnp.int32),
            pltpu.VMEM((rows_per_w, EMBED), jnp.float32),
            pltpu.VMEM((rows_per_w * EMBED,), jnp.float32),
            pltpu.SemaphoreType.DMA,
            pltpu.SemaphoreType.DMA,
            pltpu.SemaphoreType.DMA,
            pltpu.SemaphoreType.DMA,
        ],
    )
    def sc_kernel(x_hbm, table_hbm, poe_hbm, out_hbm,
                  idx_v, rows_v, poe_v, sem_a, sem_b, sem_p, sem_o):
        wid = lax.axis_index("s") * num_cores + lax.axis_index("c")
        base = wid * rows_per_w
        half = rows_per_w // 2
        chunk = 32
        n_chunks = rows_per_w // chunk

        pltpu.sync_copy(x_hbm.at[pl.ds(base, rows_per_w)], idx_v)
        g_a = pltpu.async_copy(table_hbm.at[idx_v.at[pl.ds(0, half)]],
                               rows_v.at[pl.ds(0, half), :], sem_a)
        g_b = pltpu.async_copy(table_hbm.at[idx_v.at[pl.ds(half, half)]],
                               rows_v.at[pl.ds(half, half), :], sem_b)
        g_p = pltpu.async_copy(
            poe_hbm.at[pl.ds(base * EMBED, rows_per_w * EMBED)], poe_v, sem_p)

        def norm_rows(i, c0):
            outs = []
            for u in range(UNROLL):
                r = c0 + i * UNROLL + u
                xs = []
                for c in range(CHUNKS):
                    xs.append(rows_v[r, pl.ds(c * LANES, LANES)]
                              + poe_v[pl.ds(r * EMBED + c * LANES, LANES)])
                s = xs[0]
                q = xs[0] * xs[0]
                for c in range(1, CHUNKS):
                    s = s + xs[c]
                    q = q + xs[c] * xs[c]
                outs.append((r, xs, s, q))
            for r, xs, s, q in outs:
                mean = _hsum(s) * (1.0 / EMBED)
                var = _hsum(q) * (1.0 / EMBED) - mean * mean
                inv = _rsqrt_newton(var + 1e-5)
                for c in range(CHUNKS):
                    rows_v[r, pl.ds(c * LANES, LANES)] = (xs[c] - mean) * inv
            return c0

        g_p.wait()
        out_copies = []
        for c in range(n_chunks):
            if c == 0:
                g_a.wait()
            if c == n_chunks // 2:
                g_b.wait()
            lax.fori_loop(0, chunk // UNROLL, norm_rows, c * chunk)
            for t in range(T):
                out_copies.append(pltpu.async_copy(
                    rows_v.at[pl.ds(c * chunk, chunk), :],
                    out_hbm.at[t, pl.ds(base + c * chunk, chunk), :], sem_o))
        for d in out_copies:
            d.wait()

    return sc_kernel


def kernel(x, cur_pos, emb_table):
    seq = x.shape[0]
    info = plsc.get_sparse_core_info()
    n_workers = info.num_cores * info.num_subcores
    rows_per_w = seq // n_workers
    poe = _poe(seq, EMBED)
    sc = _make_sc_kernel(seq, info.num_cores, rows_per_w)
    return sc(x, emb_table, poe)


# speedup vs baseline: 1.0047x; 1.0047x over previous
import functools

import jax
import jax.numpy as jnp
import numpy as np
from jax import lax
from jax.experimental import pallas as pl
from jax.experimental.pallas import tpu as pltpu
from jax.experimental.pallas import tpu_sc as plsc

EMBED = 128
T = 4
LANES = 16
CHUNKS = EMBED // LANES
UNROLL = 4


@functools.lru_cache(maxsize=None)
def _poe_factors(ctx, emb):
    nlo = 64
    nhi = ctx // nlo
    j = np.arange(emb)
    w = 10000.0 ** (-(j - (j % 2)) / emb)
    even = (j % 2 == 0)
    ah = (np.arange(nhi)[:, None] * nlo) * w[None, :]
    al = np.arange(nlo)[:, None] * w[None, :]
    u = np.where(even, np.sin(ah), np.cos(ah))
    v = np.where(even, np.cos(ah), -np.sin(ah))
    cl = np.cos(al)
    sl = np.sin(al)
    f32 = lambda a: jnp.asarray(a.astype(np.float32))
    return f32(u), f32(v), f32(cl), f32(sl)


def _poe(ctx, emb):
    u, v, cl, sl = _poe_factors(ctx, emb)
    poe3 = u[:, None, :] * cl[None, :, :] + v[:, None, :] * sl[None, :, :]
    return poe3.reshape(ctx * emb)


def _rsqrt_newton(v):
    i = lax.bitcast_convert_type(v, jnp.int32)
    i = jnp.int32(0x5F3759DF) - lax.shift_right_logical(i, 1)
    y = lax.bitcast_convert_type(i, jnp.float32)
    y = y * (1.5 - 0.5 * v * y * y)
    return y


def _shuffle(v, k):
    iota = lax.iota(jnp.int32, LANES)
    dnums = lax.GatherDimensionNumbers(
        offset_dims=(), collapsed_slice_dims=(0,), start_index_map=(0,))
    idx = jnp.bitwise_xor(iota, k)
    return lax.gather(v, idx[:, None], dimension_numbers=dnums,
                      slice_sizes=(1,),
                      mode=lax.GatherScatterMode.PROMISE_IN_BOUNDS)


def _hsum(v):
    for k in (8, 4, 2, 1):
        v = v + _shuffle(v, k)
    return v


def _make_sc_kernel(seq, num_cores, rows_per_w):
    mesh = plsc.VectorSubcoreMesh(core_axis_name="c", subcore_axis_name="s")

    @functools.partial(
        pl.kernel,
        mesh=mesh,
        out_type=jax.ShapeDtypeStruct((T, seq, EMBED), jnp.float32),
        scratch_types=[
            pltpu.VMEM((rows_per_w,), jnp.int32),
            pltpu.VMEM((rows_per_w, EMBED), jnp.float32),
            pltpu.VMEM((rows_per_w * EMBED,), jnp.float32),
            pltpu.SemaphoreType.DMA,
            pltpu.SemaphoreType.DMA,
            pltpu.SemaphoreType.DMA,
            pltpu.SemaphoreType.DMA,
        ],
    )
    def sc_kernel(x_hbm, table_hbm, poe_hbm, out_hbm,
                  idx_v, rows_v, poe_v, sem_a, sem_b, sem_p, sem_o):
        wid = lax.axis_index("s") * num_cores + lax.axis_index("c")
        base = wid * rows_per_w
        half = rows_per_w // 2
        chunk = 32
        n_chunks = rows_per_w // chunk

        pltpu.sync_copy(x_hbm.at[pl.ds(base, rows_per_w)], idx_v)
        g_a = pltpu.async_copy(table_hbm.at[idx_v.at[pl.ds(0, half)]],
                               rows_v.at[pl.ds(0, half), :], sem_a)
        g_b = pltpu.async_copy(table_hbm.at[idx_v.at[pl.ds(half, half)]],
                               rows_v.at[pl.ds(half, half), :], sem_b)
        g_p = pltpu.async_copy(
            poe_hbm.at[pl.ds(base * EMBED, rows_per_w * EMBED)], poe_v, sem_p)

        def norm_rows(i, c0):
            outs = []
            for u in range(UNROLL):
                r = c0 + i * UNROLL + u
                xs = []
                for c in range(CHUNKS):
                    xs.append(rows_v[r, pl.ds(c * LANES, LANES)]
                              + poe_v[pl.ds(r * EMBED + c * LANES, LANES)])
                s = xs[0]
                q = xs[0] * xs[0]
                for c in range(1, CHUNKS):
                    s = s + xs[c]
                    q = q + xs[c] * xs[c]
                outs.append((r, xs, s, q))
            for r, xs, s, q in outs:
                mean = _hsum(s) * (1.0 / EMBED)
                var = _hsum(q) * (1.0 / EMBED) - mean * mean
                inv = _rsqrt_newton(var + 1e-5)
                for c in range(CHUNKS):
                    rows_v[r, pl.ds(c * LANES, LANES)] = (xs[c] - mean) * inv
            return c0

        g_p.wait()
        out_copies = []
        for c in range(n_chunks):
            if c == 0:
                g_a.wait()
            if c == n_chunks // 2:
                g_b.wait()
            lax.fori_loop(0, chunk // UNROLL, norm_rows, c * chunk)
            for t in range(T):
                out_copies.append(pltpu.async_copy(
                    rows_v.at[pl.ds(c * chunk, chunk), :],
                    out_hbm.at[t, pl.ds(base + c * chunk, chunk), :], sem_o))
        for d in out_copies:
            d.wait()

    return sc_kernel


def kernel(x, cur_pos, emb_table):
    seq = x.shape[0]
    info = plsc.get_sparse_core_info()
    n_workers = info.num_cores * info.num_subcores
    rows_per_w = seq // n_workers
    poe = _poe(seq, EMBED)
    sc = _make_sc_kernel(seq, info.num_cores, rows_per_w)
    return sc(x, emb_table, poe)
